# baseline (speedup 1.0000x reference)
import functools

import jax
import jax.numpy as jnp
from jax import lax
from jax.experimental import pallas as pl
from jax.experimental.pallas import tpu as pltpu
from jax.experimental.pallas import tpu_sc as plsc

_TOKENS = 4096
_D = 768
_H = 64
_E = 8
_BLK = 512
_NBLK = _TOKENS // _BLK
_NW = 32
_CHUNK = _TOKENS // _NW
_LANES = 16
_SGW = 128


def _router_body(x_ref, ws1_ref, bs1_ref, w2a_ref, b2a_ref, w2b_ref, b2b_ref,
                 w2c_ref, b2c_ref, g_ref, eps_ref,
                 ylog_ref, yidx_ref, y_ref, zm_ref, zlv_ref, z_ref,
                 rank_ref, cnt_ref, start_ref, sgate_ref, base_ref):
    i = pl.program_id(0)

    @pl.when(i == 0)
    def _init():
        base_ref[...] = jnp.zeros((1, _LANES), jnp.float32)

    xb = x_ref[...]
    h = jnp.maximum(jnp.dot(xb, ws1_ref[...]) + bs1_ref[...], 0.0)
    ylog = jnp.dot(h, w2a_ref[...]) + b2a_ref[...]
    zm = jnp.dot(h, w2b_ref[...]) + b2b_ref[...]
    zlv = jnp.dot(h, w2c_ref[...]) + b2c_ref[...]
    a = ylog + g_ref[...]
    m = jnp.max(a, axis=1, keepdims=True)
    ex = jnp.exp(a - m)
    ysoft = ex / jnp.sum(ex, axis=1, keepdims=True)
    lane = lax.broadcasted_iota(jnp.int32, (_BLK, _E), 1)
    yidx = jnp.min(jnp.where(a == m, lane, _E), axis=1)
    yhard = (lane == yidx[:, None]).astype(jnp.float32)
    y = ysoft + (yhard - ysoft)
    z = zm + jnp.exp(0.5 * zlv) * eps_ref[...]

    lane16 = lax.broadcasted_iota(jnp.int32, (_BLK, _LANES), 1)
    hot16 = (lane16 == yidx[:, None]).astype(jnp.float32)
    r_io = lax.broadcasted_iota(jnp.int32, (_BLK, _BLK), 0)
    c_io = lax.broadcasted_iota(jnp.int32, (_BLK, _BLK), 1)
    tri = (r_io >= c_io).astype(jnp.float32)
    cum = jnp.dot(tri, hot16, precision=lax.Precision.HIGHEST)
    base = base_ref[...]
    rank = jnp.sum((cum - hot16 + base) * hot16, axis=1)
    base = base + cum[_BLK - 1:_BLK, :]
    base_ref[...] = base
    cnt_ref[...] = base.astype(jnp.int32)
    r16 = lax.broadcasted_iota(jnp.int32, (_LANES, _LANES), 0)
    c16 = lax.broadcasted_iota(jnp.int32, (_LANES, _LANES), 1)
    sut = (r16 < c16).astype(jnp.float32)
    starts = jnp.dot(base, sut, precision=lax.Precision.HIGHEST)
    start_ref[...] = starts.astype(jnp.int32)
    s = jnp.sum(yhard * z, axis=1)

    ylog_ref[...] = ylog
    yidx_ref[...] = yidx[:, None]
    y_ref[...] = y
    zm_ref[...] = zm
    zlv_ref[...] = zlv
    z_ref[...] = z
    rank_ref[...] = rank.astype(jnp.int32)[:, None]
    sgate_ref[...] = jnp.broadcast_to(s[:, None], (_BLK, _SGW))


_SC_MESH = plsc.VectorSubcoreMesh(core_axis_name="c", subcore_axis_name="s")


def _sc_wid():
    return lax.axis_index("s") * 2 + lax.axis_index("c")


@functools.partial(
    pl.kernel,
    out_type=[
        jax.ShapeDtypeStruct((_TOKENS, _D), jnp.float32),
        jax.ShapeDtypeStruct((_TOKENS, _SGW), jnp.float32),
        jax.ShapeDtypeStruct((_TOKENS,), jnp.int32),
    ],
    mesh=_SC_MESH,
    scratch_types=[
        pltpu.VMEM((_CHUNK,), jnp.int32),
        pltpu.VMEM((_CHUNK,), jnp.int32),
        pltpu.VMEM((_LANES,), jnp.int32),
        pltpu.VMEM((_CHUNK,), jnp.int32),
        pltpu.VMEM((_CHUNK, _D), jnp.float32),
        pltpu.VMEM((_CHUNK, _SGW), jnp.float32),
    ],
    compiler_params=pltpu.CompilerParams(needs_layout_passes=False),
)
def _dispatch(x_hbm, rank_hbm, eid_hbm, start_hbm, sg_hbm,
              xs_hbm, ss_hbm, pos_hbm,
              rank_v, eid_v, cnt_v, pos_v, rows_v, sg_v):
    base = _sc_wid() * _CHUNK
    pltpu.sync_copy(start_hbm, cnt_v)
    pltpu.sync_copy(rank_hbm.at[pl.ds(base, _CHUNK)], rank_v)
    pltpu.sync_copy(eid_hbm.at[pl.ds(base, _CHUNK)], eid_v)
    idx16 = lax.iota(jnp.int32, _LANES)
    sv = cnt_v[...]
    st = [jnp.sum(jnp.where(idx16 == e, sv, 0)) for e in range(_E)]
    for j in range(_CHUNK // _LANES):
        sl = pl.ds(j * _LANES, _LANES)
        ev = eid_v[sl]
        acc = jnp.full((_LANES,), 0, jnp.int32)
        for e in range(_E):
            acc = jnp.where(ev == e, st[e], acc)
        pos_v[sl] = rank_v[sl] + acc
    pltpu.sync_copy(pos_v, pos_hbm.at[pl.ds(base, _CHUNK)])
    pltpu.sync_copy(x_hbm.at[pl.ds(base, _CHUNK)], rows_v)
    pltpu.sync_copy(sg_hbm.at[pl.ds(base, _CHUNK)], sg_v)
    pltpu.sync_copy(rows_v, xs_hbm.at[pos_v])
    pltpu.sync_copy(sg_v, ss_hbm.at[pos_v])


@functools.partial(
    pl.kernel,
    out_type=jax.ShapeDtypeStruct((_TOKENS, _D), jnp.float32),
    mesh=_SC_MESH,
    scratch_types=[
        pltpu.VMEM((_CHUNK,), jnp.int32),
        pltpu.VMEM((_CHUNK, _D), jnp.float32),
        pltpu.SemaphoreType.DMA,
    ],
    compiler_params=pltpu.CompilerParams(needs_layout_passes=False),
)
def _combine(os_hbm, pos_hbm, out_hbm, pos_v, rows_v, sem):
    base = _sc_wid() * _CHUNK
    pltpu.sync_copy(pos_hbm.at[pl.ds(base, _CHUNK)], pos_v)
    pltpu.async_copy(os_hbm.at[pos_v], rows_v, sem).wait()
    pltpu.sync_copy(rows_v, out_hbm.at[pl.ds(base, _CHUNK)])


def _expert_body(cnt_ref, xs_ref, ss_ref, w1_ref, b1_ref, w2_ref, b2_ref,
                 out_ref):
    i = pl.program_id(0)
    xb = xs_ref[...]
    out_ref[...] = xb
    sc_col = ss_ref[...][:, 0:1]
    rows = i * _BLK + lax.broadcasted_iota(jnp.int32, (_BLK, 1), 0)
    start = jnp.int32(0)
    for e in range(_E):
        cnt_e = cnt_ref[0, e]
        lo = start
        pred = jnp.logical_and(lo < (i + 1) * _BLK, lo + cnt_e > i * _BLK)

        @pl.when(pred)
        def _run(lo=lo, cnt_e=cnt_e, e=e):
            m = jnp.logical_and(rows >= lo, rows < lo + cnt_e)
            g = m.astype(jnp.float32) * sc_col
            hb = jnp.dot(xb.astype(jnp.bfloat16), w1_ref[e],
                         preferred_element_type=jnp.float32)
            hb = jnp.maximum(hb + b1_ref[e][None, :], 0.0)
            hg = (hb * g).astype(jnp.bfloat16)
            ob = jnp.dot(hg, w2_ref[e], preferred_element_type=jnp.float32)
            out_ref[...] += ob + g * b2_ref[e][None, :]

        start = start + cnt_e


def _noise():
    def build():
        key = jax.random.key(42)
        k1, k2 = jax.random.split(key)
        u = jax.random.uniform(k1, (_TOKENS, _E), minval=1e-6, maxval=1.0 - 1e-6)
        g = -jnp.log(-jnp.log(u))
        eps = jax.random.normal(k2, (_TOKENS, _E), dtype=jnp.float32)
        return g, eps
    try:
        with jax.ensure_compile_time_eval():
            return build()
    except Exception:
        return build()


@jax.jit
def kernel(x, Ws1, bs1, Ws2, bs2, W1, b1, W2, b2):
    g, eps = _noise()

    w2a, w2b, w2c = Ws2[:, 0:_E], Ws2[:, _E:2 * _E], Ws2[:, 2 * _E:3 * _E]
    b2a, b2b, b2c = (bs2[0:_E].reshape(1, _E), bs2[_E:2 * _E].reshape(1, _E),
                     bs2[2 * _E:3 * _E].reshape(1, _E))
    bs1r = bs1.reshape(1, _H)
    w1bf = W1.astype(jnp.bfloat16)
    w2bf = W2.astype(jnp.bfloat16)

    row = lambda w: pl.BlockSpec((_BLK, w), lambda i: (i, 0))
    full = lambda arr: pl.BlockSpec(arr.shape, lambda i: (0,) * arr.ndim)

    router_outs = pl.pallas_call(
        _router_body,
        grid=(_NBLK,),
        in_specs=[
            row(_D), full(Ws1), full(bs1r),
            full(w2a), full(b2a), full(w2b), full(b2b), full(w2c), full(b2c),
            row(_E), row(_E),
        ],
        out_specs=[
            row(_E), row(1), row(_E), row(_E), row(_E), row(_E),
            row(1), pl.BlockSpec((1, _LANES), lambda i: (0, 0)),
            pl.BlockSpec((1, _LANES), lambda i: (0, 0)), row(_SGW),
        ],
        out_shape=[
            jax.ShapeDtypeStruct((_TOKENS, _E), jnp.float32),
            jax.ShapeDtypeStruct((_TOKENS, 1), jnp.int32),
            jax.ShapeDtypeStruct((_TOKENS, _E), jnp.float32),
            jax.ShapeDtypeStruct((_TOKENS, _E), jnp.float32),
            jax.ShapeDtypeStruct((_TOKENS, _E), jnp.float32),
            jax.ShapeDtypeStruct((_TOKENS, _E), jnp.float32),
            jax.ShapeDtypeStruct((_TOKENS, 1), jnp.int32),
            jax.ShapeDtypeStruct((1, _LANES), jnp.int32),
            jax.ShapeDtypeStruct((1, _LANES), jnp.int32),
            jax.ShapeDtypeStruct((_TOKENS, _SGW), jnp.float32),
        ],
        scratch_shapes=[pltpu.VMEM((1, _LANES), jnp.float32)],
    )(x, Ws1, bs1r, w2a, b2a, w2b, b2b, w2c, b2c, g, eps)

    ylog, yidx2, y, zm, zlv, z, rank2, cnt2, start2, sgate = router_outs
    rank = rank2.reshape(_TOKENS)
    eid = yidx2.reshape(_TOKENS)
    starts1 = start2.reshape(_LANES)

    xs, ss, pos = _dispatch(x, rank, eid, starts1, sgate)

    out_sorted = pl.pallas_call(
        _expert_body,
        grid=(_NBLK,),
        in_specs=[
            pl.BlockSpec(memory_space=pltpu.SMEM),
            row(_D), row(_SGW),
            full(w1bf), full(b1), full(w2bf), full(b2),
        ],
        out_specs=row(_D),
        out_shape=jax.ShapeDtypeStruct((_TOKENS, _D), jnp.float32),
    )(cnt2, xs, ss, w1bf, b1, w2bf, b2)

    out = _combine(out_sorted, pos)
    return (out, ylog, eid, y, zm, zlv, z)


# speedup vs baseline: 1.0701x; 1.0701x over previous
import functools

import jax
import jax.numpy as jnp
from jax import lax
from jax.experimental import pallas as pl
from jax.experimental.pallas import tpu as pltpu
from jax.experimental.pallas import tpu_sc as plsc

_TOKENS = 4096
_D = 768
_H = 64
_E = 8
_BLK = 512
_NBLK = _TOKENS // _BLK
_NW = 32
_CHUNK = _TOKENS // _NW
_LANES = 16
_SGW = 128


def _router_body(x_ref, ws1_ref, bs1_ref, w2a_ref, b2a_ref, w2b_ref, b2b_ref,
                 w2c_ref, b2c_ref, g_ref, eps_ref,
                 ylog_ref, yidx_ref, y_ref, zm_ref, zlv_ref, z_ref,
                 rank_ref, cnt_ref, start_ref, sgate_ref, base_ref):
    i = pl.program_id(0)

    @pl.when(i == 0)
    def _init():
        base_ref[...] = jnp.zeros((1, _LANES), jnp.float32)

    xb = x_ref[...]
    h = jnp.maximum(jnp.dot(xb, ws1_ref[...]) + bs1_ref[...], 0.0)
    ylog = jnp.dot(h, w2a_ref[...]) + b2a_ref[...]
    zm = jnp.dot(h, w2b_ref[...]) + b2b_ref[...]
    zlv = jnp.dot(h, w2c_ref[...]) + b2c_ref[...]
    a = ylog + g_ref[...]
    m = jnp.max(a, axis=1, keepdims=True)
    ex = jnp.exp(a - m)
    ysoft = ex / jnp.sum(ex, axis=1, keepdims=True)
    lane = lax.broadcasted_iota(jnp.int32, (_BLK, _E), 1)
    yidx = jnp.min(jnp.where(a == m, lane, _E), axis=1)
    yhard = (lane == yidx[:, None]).astype(jnp.float32)
    y = ysoft + (yhard - ysoft)
    z = zm + jnp.exp(0.5 * zlv) * eps_ref[...]

    lane16 = lax.broadcasted_iota(jnp.int32, (_BLK, _LANES), 1)
    hot16 = (lane16 == yidx[:, None]).astype(jnp.float32)
    r_io = lax.broadcasted_iota(jnp.int32, (_BLK, _BLK), 0)
    c_io = lax.broadcasted_iota(jnp.int32, (_BLK, _BLK), 1)
    tri = (r_io >= c_io).astype(jnp.bfloat16)
    cum = jnp.dot(tri, hot16.astype(jnp.bfloat16),
                  preferred_element_type=jnp.float32)
    base = base_ref[...]
    rank = jnp.sum((cum - hot16 + base) * hot16, axis=1)
    base = base + cum[_BLK - 1:_BLK, :]
    base_ref[...] = base
    cnt_ref[...] = base.astype(jnp.int32)
    r16 = lax.broadcasted_iota(jnp.int32, (_LANES, _LANES), 0)
    c16 = lax.broadcasted_iota(jnp.int32, (_LANES, _LANES), 1)
    sut = (r16 < c16).astype(jnp.float32)
    starts = jnp.dot(base, sut, precision=lax.Precision.HIGHEST)
    start_ref[...] = starts.astype(jnp.int32)
    s = jnp.sum(yhard * z, axis=1)

    ylog_ref[...] = ylog
    yidx_ref[...] = yidx[:, None]
    y_ref[...] = y
    zm_ref[...] = zm
    zlv_ref[...] = zlv
    z_ref[...] = z
    rank_ref[...] = rank.astype(jnp.int32)[:, None]
    sgate_ref[...] = jnp.broadcast_to(s[:, None], (_BLK, _SGW))


_SC_MESH = plsc.VectorSubcoreMesh(core_axis_name="c", subcore_axis_name="s")


def _sc_wid():
    return lax.axis_index("s") * 2 + lax.axis_index("c")


@functools.partial(
    pl.kernel,
    out_type=[
        jax.ShapeDtypeStruct((_TOKENS, _D), jnp.float32),
        jax.ShapeDtypeStruct((_TOKENS, _SGW), jnp.float32),
        jax.ShapeDtypeStruct((_TOKENS,), jnp.int32),
    ],
    mesh=_SC_MESH,
    scratch_types=[
        pltpu.VMEM((_CHUNK,), jnp.int32),
        pltpu.VMEM((_CHUNK,), jnp.int32),
        pltpu.VMEM((_LANES,), jnp.int32),
        pltpu.VMEM((_CHUNK,), jnp.int32),
        pltpu.VMEM((_CHUNK, _D), jnp.float32),
        pltpu.VMEM((_CHUNK, _SGW), jnp.float32),
        pltpu.SemaphoreType.DMA,
        pltpu.SemaphoreType.DMA,
    ],
    compiler_params=pltpu.CompilerParams(needs_layout_passes=False),
)
def _dispatch(x_hbm, rank_hbm, eid_hbm, start_hbm, sg_hbm,
              xs_hbm, ss_hbm, pos_hbm,
              rank_v, eid_v, cnt_v, pos_v, rows_v, sg_v, sem_x, sem_sg):
    base = _sc_wid() * _CHUNK
    cp_x = pltpu.async_copy(x_hbm.at[pl.ds(base, _CHUNK)], rows_v, sem_x)
    cp_sg = pltpu.async_copy(sg_hbm.at[pl.ds(base, _CHUNK)], sg_v, sem_sg)
    pltpu.sync_copy(start_hbm, cnt_v)
    pltpu.sync_copy(rank_hbm.at[pl.ds(base, _CHUNK)], rank_v)
    pltpu.sync_copy(eid_hbm.at[pl.ds(base, _CHUNK)], eid_v)
    idx16 = lax.iota(jnp.int32, _LANES)
    sv = cnt_v[...]
    st = [jnp.sum(jnp.where(idx16 == e, sv, 0)) for e in range(_E)]
    for j in range(_CHUNK // _LANES):
        sl = pl.ds(j * _LANES, _LANES)
        ev = eid_v[sl]
        acc = jnp.full((_LANES,), 0, jnp.int32)
        for e in range(_E):
            acc = jnp.where(ev == e, st[e], acc)
        pos_v[sl] = rank_v[sl] + acc
    pltpu.sync_copy(pos_v, pos_hbm.at[pl.ds(base, _CHUNK)])
    cp_x.wait()
    pltpu.sync_copy(rows_v, xs_hbm.at[pos_v])
    cp_sg.wait()
    pltpu.sync_copy(sg_v, ss_hbm.at[pos_v])


@functools.partial(
    pl.kernel,
    out_type=jax.ShapeDtypeStruct((_TOKENS, _D), jnp.float32),
    mesh=_SC_MESH,
    scratch_types=[
        pltpu.VMEM((_CHUNK,), jnp.int32),
        pltpu.VMEM((_CHUNK, _D), jnp.float32),
        pltpu.SemaphoreType.DMA,
    ],
    compiler_params=pltpu.CompilerParams(needs_layout_passes=False),
)
def _combine(os_hbm, pos_hbm, out_hbm, pos_v, rows_v, sem):
    base = _sc_wid() * _CHUNK
    pltpu.sync_copy(pos_hbm.at[pl.ds(base, _CHUNK)], pos_v)
    pltpu.async_copy(os_hbm.at[pos_v], rows_v, sem).wait()
    pltpu.sync_copy(rows_v, out_hbm.at[pl.ds(base, _CHUNK)])


def _expert_body(cnt_ref, xs_ref, ss_ref, w1_ref, b1_ref, w2_ref, b2_ref,
                 out_ref):
    i = pl.program_id(0)
    xb = xs_ref[...]
    out_ref[...] = xb
    sc_col = ss_ref[...][:, 0:1]
    rows = i * _BLK + lax.broadcasted_iota(jnp.int32, (_BLK, 1), 0)
    start = jnp.int32(0)
    for e in range(_E):
        cnt_e = cnt_ref[0, e]
        lo = start
        pred = jnp.logical_and(lo < (i + 1) * _BLK, lo + cnt_e > i * _BLK)

        @pl.when(pred)
        def _run(lo=lo, cnt_e=cnt_e, e=e):
            m = jnp.logical_and(rows >= lo, rows < lo + cnt_e)
            g = m.astype(jnp.float32) * sc_col
            hb = jnp.dot(xb.astype(jnp.bfloat16), w1_ref[e],
                         preferred_element_type=jnp.float32)
            hb = jnp.maximum(hb + b1_ref[e][None, :], 0.0)
            hg = (hb * g).astype(jnp.bfloat16)
            ob = jnp.dot(hg, w2_ref[e], preferred_element_type=jnp.float32)
            out_ref[...] += ob + g * b2_ref[e][None, :]

        start = start + cnt_e


def _noise():
    def build():
        key = jax.random.key(42)
        k1, k2 = jax.random.split(key)
        u = jax.random.uniform(k1, (_TOKENS, _E), minval=1e-6, maxval=1.0 - 1e-6)
        g = -jnp.log(-jnp.log(u))
        eps = jax.random.normal(k2, (_TOKENS, _E), dtype=jnp.float32)
        return g, eps
    try:
        with jax.ensure_compile_time_eval():
            return build()
    except Exception:
        return build()


@jax.jit
def kernel(x, Ws1, bs1, Ws2, bs2, W1, b1, W2, b2):
    g, eps = _noise()

    w2a, w2b, w2c = Ws2[:, 0:_E], Ws2[:, _E:2 * _E], Ws2[:, 2 * _E:3 * _E]
    b2a, b2b, b2c = (bs2[0:_E].reshape(1, _E), bs2[_E:2 * _E].reshape(1, _E),
                     bs2[2 * _E:3 * _E].reshape(1, _E))
    bs1r = bs1.reshape(1, _H)
    w1bf = W1.astype(jnp.bfloat16)
    w2bf = W2.astype(jnp.bfloat16)

    row = lambda w: pl.BlockSpec((_BLK, w), lambda i: (i, 0))
    full = lambda arr: pl.BlockSpec(arr.shape, lambda i: (0,) * arr.ndim)

    router_outs = pl.pallas_call(
        _router_body,
        grid=(_NBLK,),
        in_specs=[
            row(_D), full(Ws1), full(bs1r),
            full(w2a), full(b2a), full(w2b), full(b2b), full(w2c), full(b2c),
            row(_E), row(_E),
        ],
        out_specs=[
            row(_E), row(1), row(_E), row(_E), row(_E), row(_E),
            row(1), pl.BlockSpec((1, _LANES), lambda i: (0, 0)),
            pl.BlockSpec((1, _LANES), lambda i: (0, 0)), row(_SGW),
        ],
        out_shape=[
            jax.ShapeDtypeStruct((_TOKENS, _E), jnp.float32),
            jax.ShapeDtypeStruct((_TOKENS, 1), jnp.int32),
            jax.ShapeDtypeStruct((_TOKENS, _E), jnp.float32),
            jax.ShapeDtypeStruct((_TOKENS, _E), jnp.float32),
            jax.ShapeDtypeStruct((_TOKENS, _E), jnp.float32),
            jax.ShapeDtypeStruct((_TOKENS, _E), jnp.float32),
            jax.ShapeDtypeStruct((_TOKENS, 1), jnp.int32),
            jax.ShapeDtypeStruct((1, _LANES), jnp.int32),
            jax.ShapeDtypeStruct((1, _LANES), jnp.int32),
            jax.ShapeDtypeStruct((_TOKENS, _SGW), jnp.float32),
        ],
        scratch_shapes=[pltpu.VMEM((1, _LANES), jnp.float32)],
    )(x, Ws1, bs1r, w2a, b2a, w2b, b2b, w2c, b2c, g, eps)

    ylog, yidx2, y, zm, zlv, z, rank2, cnt2, start2, sgate = router_outs
    rank = rank2.reshape(_TOKENS)
    eid = yidx2.reshape(_TOKENS)
    starts1 = start2.reshape(_LANES)

    xs, ss, pos = _dispatch(x, rank, eid, starts1, sgate)

    out_sorted = pl.pallas_call(
        _expert_body,
        grid=(_NBLK,),
        in_specs=[
            pl.BlockSpec(memory_space=pltpu.SMEM),
            row(_D), row(_SGW),
            full(w1bf), full(b1), full(w2bf), full(b2),
        ],
        out_specs=row(_D),
        out_shape=jax.ShapeDtypeStruct((_TOKENS, _D), jnp.float32),
    )(cnt2, xs, ss, w1bf, b1, w2bf, b2)

    out = _combine(out_sorted, pos)
    return (out, ylog, eid, y, zm, zlv, z)


# speedup vs baseline: 1.1052x; 1.0327x over previous
import functools

import jax
import jax.numpy as jnp
from jax import lax
from jax.experimental import pallas as pl
from jax.experimental.pallas import tpu as pltpu
from jax.experimental.pallas import tpu_sc as plsc

_TOKENS = 4096
_D = 768
_H = 64
_E = 8
_BLK = 512
_NBLK = _TOKENS // _BLK
_NW = 32
_CHUNK = _TOKENS // _NW
_LANES = 16
_SGW = 128
_DP = _D // 2


def _router_body(x_ref, ws1_ref, bs1_ref, w2a_ref, b2a_ref, w2b_ref, b2b_ref,
                 w2c_ref, b2c_ref, g_ref, eps_ref,
                 ylog_ref, yidx_ref, y_ref, zm_ref, zlv_ref, z_ref,
                 rank_ref, cnt_ref, start_ref, sgate_ref, xp_ref, base_ref):
    i = pl.program_id(0)

    @pl.when(i == 0)
    def _init():
        base_ref[...] = jnp.zeros((1, _LANES), jnp.float32)

    xb = x_ref[...]
    h = jnp.maximum(jnp.dot(xb, ws1_ref[...]) + bs1_ref[...], 0.0)
    ylog = jnp.dot(h, w2a_ref[...]) + b2a_ref[...]
    zm = jnp.dot(h, w2b_ref[...]) + b2b_ref[...]
    zlv = jnp.dot(h, w2c_ref[...]) + b2c_ref[...]
    a = ylog + g_ref[...]
    m = jnp.max(a, axis=1, keepdims=True)
    ex = jnp.exp(a - m)
    ysoft = ex / jnp.sum(ex, axis=1, keepdims=True)
    lane = lax.broadcasted_iota(jnp.int32, (_BLK, _E), 1)
    yidx = jnp.min(jnp.where(a == m, lane, _E), axis=1)
    yhard = (lane == yidx[:, None]).astype(jnp.float32)
    y = ysoft + (yhard - ysoft)
    z = zm + jnp.exp(0.5 * zlv) * eps_ref[...]

    lane16 = lax.broadcasted_iota(jnp.int32, (_BLK, _LANES), 1)
    hot16 = (lane16 == yidx[:, None]).astype(jnp.float32)
    r_io = lax.broadcasted_iota(jnp.int32, (_BLK, _BLK), 0)
    c_io = lax.broadcasted_iota(jnp.int32, (_BLK, _BLK), 1)
    tri = (r_io >= c_io).astype(jnp.bfloat16)
    cum = jnp.dot(tri, hot16.astype(jnp.bfloat16),
                  preferred_element_type=jnp.float32)
    base = base_ref[...]
    rank = jnp.sum((cum - hot16 + base) * hot16, axis=1)
    base = base + cum[_BLK - 1:_BLK, :]
    base_ref[...] = base
    cnt_ref[...] = base.astype(jnp.int32)
    r16 = lax.broadcasted_iota(jnp.int32, (_LANES, _LANES), 0)
    c16 = lax.broadcasted_iota(jnp.int32, (_LANES, _LANES), 1)
    sut = (r16 < c16).astype(jnp.float32)
    starts = jnp.dot(base, sut, precision=lax.Precision.HIGHEST)
    start_ref[...] = starts.astype(jnp.int32)
    s = jnp.sum(yhard * z, axis=1)

    ylog_ref[...] = ylog
    yidx_ref[...] = yidx[:, None]
    y_ref[...] = y
    zm_ref[...] = zm
    zlv_ref[...] = zlv
    z_ref[...] = z
    rank_ref[...] = rank.astype(jnp.int32)[:, None]
    sgate_ref[...] = jnp.broadcast_to(s[:, None], (_BLK, _SGW))
    xr = lax.bitcast_convert_type(
        xb.astype(jnp.bfloat16).astype(jnp.float32), jnp.uint32)
    xp_ref[...] = (xr[:, :_DP] >> 16) | (xr[:, _DP:] & jnp.uint32(0xFFFF0000))


_SC_MESH = plsc.VectorSubcoreMesh(core_axis_name="c", subcore_axis_name="s")


def _sc_wid():
    return lax.axis_index("s") * 2 + lax.axis_index("c")


@functools.partial(
    pl.kernel,
    out_type=[
        jax.ShapeDtypeStruct((_TOKENS, _DP), jnp.uint32),
        jax.ShapeDtypeStruct((_TOKENS, _SGW), jnp.float32),
        jax.ShapeDtypeStruct((_TOKENS,), jnp.int32),
    ],
    mesh=_SC_MESH,
    scratch_types=[
        pltpu.VMEM((_CHUNK,), jnp.int32),
        pltpu.VMEM((_CHUNK,), jnp.int32),
        pltpu.VMEM((_LANES,), jnp.int32),
        pltpu.VMEM((_CHUNK,), jnp.int32),
        pltpu.VMEM((_CHUNK, _DP), jnp.uint32),
        pltpu.VMEM((_CHUNK, _SGW), jnp.float32),
        pltpu.SemaphoreType.DMA,
        pltpu.SemaphoreType.DMA,
    ],
    compiler_params=pltpu.CompilerParams(needs_layout_passes=False),
)
def _dispatch(xp_hbm, rank_hbm, eid_hbm, start_hbm, sg_hbm,
              xsp_hbm, ss_hbm, pos_hbm,
              rank_v, eid_v, cnt_v, pos_v, rows_v, sg_v, sem_x, sem_sg):
    base = _sc_wid() * _CHUNK
    cp_x = pltpu.async_copy(xp_hbm.at[pl.ds(base, _CHUNK)], rows_v, sem_x)
    cp_sg = pltpu.async_copy(sg_hbm.at[pl.ds(base, _CHUNK)], sg_v, sem_sg)
    pltpu.sync_copy(start_hbm, cnt_v)
    pltpu.sync_copy(rank_hbm.at[pl.ds(base, _CHUNK)], rank_v)
    pltpu.sync_copy(eid_hbm.at[pl.ds(base, _CHUNK)], eid_v)
    idx16 = lax.iota(jnp.int32, _LANES)
    sv = cnt_v[...]
    st = [jnp.sum(jnp.where(idx16 == e, sv, 0)) for e in range(_E)]
    for j in range(_CHUNK // _LANES):
        sl = pl.ds(j * _LANES, _LANES)
        ev = eid_v[sl]
        acc = jnp.full((_LANES,), 0, jnp.int32)
        for e in range(_E):
            acc = jnp.where(ev == e, st[e], acc)
        pos_v[sl] = rank_v[sl] + acc
    pltpu.sync_copy(pos_v, pos_hbm.at[pl.ds(base, _CHUNK)])
    cp_x.wait()
    pltpu.sync_copy(rows_v, xsp_hbm.at[pos_v])
    cp_sg.wait()
    pltpu.sync_copy(sg_v, ss_hbm.at[pos_v])


@functools.partial(
    pl.kernel,
    out_type=jax.ShapeDtypeStruct((_TOKENS, _D), jnp.float32),
    mesh=_SC_MESH,
    scratch_types=[
        pltpu.VMEM((_CHUNK,), jnp.int32),
        pltpu.VMEM((_CHUNK, _D), jnp.float32),
        pltpu.SemaphoreType.DMA,
    ],
    compiler_params=pltpu.CompilerParams(needs_layout_passes=False),
)
def _combine(os_hbm, pos_hbm, out_hbm, pos_v, rows_v, sem):
    base = _sc_wid() * _CHUNK
    pltpu.sync_copy(pos_hbm.at[pl.ds(base, _CHUNK)], pos_v)
    pltpu.async_copy(os_hbm.at[pos_v], rows_v, sem).wait()
    pltpu.sync_copy(rows_v, out_hbm.at[pl.ds(base, _CHUNK)])


def _expert_body(cnt_ref, xsp_ref, ss_ref, w1_ref, b1_ref, w2_ref,
                 b2_ref, out_ref):
    i = pl.program_id(0)
    pi = xsp_ref[...]
    xlo = lax.bitcast_convert_type(pi << 16, jnp.float32)
    xhi = lax.bitcast_convert_type(pi & jnp.uint32(0xFFFF0000), jnp.float32)
    xf = jnp.concatenate([xlo, xhi], axis=1)
    xbf = xf.astype(jnp.bfloat16)
    out_ref[...] = xf
    sc_col = ss_ref[...][:, 0:1]
    rows = i * _BLK + lax.broadcasted_iota(jnp.int32, (_BLK, 1), 0)
    start = jnp.int32(0)
    for e in range(_E):
        cnt_e = cnt_ref[0, e]
        lo = start
        pred = jnp.logical_and(lo < (i + 1) * _BLK, lo + cnt_e > i * _BLK)

        @pl.when(pred)
        def _run(lo=lo, cnt_e=cnt_e, e=e):
            m = jnp.logical_and(rows >= lo, rows < lo + cnt_e)
            g = m.astype(jnp.float32) * sc_col
            hb = jnp.dot(xbf, w1_ref[e].astype(jnp.bfloat16),
                         preferred_element_type=jnp.float32)
            hb = jnp.maximum(hb + b1_ref[e][None, :], 0.0)
            hg = (hb * g).astype(jnp.bfloat16)
            ob = jnp.dot(hg, w2_ref[e].astype(jnp.bfloat16),
                         preferred_element_type=jnp.float32)
            out_ref[...] += ob + g * b2_ref[e][None, :]

        start = start + cnt_e


def _noise():
    def build():
        key = jax.random.key(42)
        k1, k2 = jax.random.split(key)
        u = jax.random.uniform(k1, (_TOKENS, _E), minval=1e-6, maxval=1.0 - 1e-6)
        g = -jnp.log(-jnp.log(u))
        eps = jax.random.normal(k2, (_TOKENS, _E), dtype=jnp.float32)
        return g, eps
    try:
        with jax.ensure_compile_time_eval():
            return build()
    except Exception:
        return build()


@jax.jit
def kernel(x, Ws1, bs1, Ws2, bs2, W1, b1, W2, b2):
    g, eps = _noise()

    w2a, w2b, w2c = Ws2[:, 0:_E], Ws2[:, _E:2 * _E], Ws2[:, 2 * _E:3 * _E]
    b2a, b2b, b2c = (bs2[0:_E].reshape(1, _E), bs2[_E:2 * _E].reshape(1, _E),
                     bs2[2 * _E:3 * _E].reshape(1, _E))
    bs1r = bs1.reshape(1, _H)

    row = lambda w: pl.BlockSpec((_BLK, w), lambda i: (i, 0))
    full = lambda arr: pl.BlockSpec(arr.shape, lambda i: (0,) * arr.ndim)

    router_outs = pl.pallas_call(
        _router_body,
        grid=(_NBLK,),
        in_specs=[
            row(_D), full(Ws1), full(bs1r),
            full(w2a), full(b2a), full(w2b), full(b2b), full(w2c), full(b2c),
            row(_E), row(_E),
        ],
        out_specs=[
            row(_E), row(1), row(_E), row(_E), row(_E), row(_E),
            row(1), pl.BlockSpec((1, _LANES), lambda i: (0, 0)),
            pl.BlockSpec((1, _LANES), lambda i: (0, 0)), row(_SGW),
            row(_DP),
        ],
        out_shape=[
            jax.ShapeDtypeStruct((_TOKENS, _E), jnp.float32),
            jax.ShapeDtypeStruct((_TOKENS, 1), jnp.int32),
            jax.ShapeDtypeStruct((_TOKENS, _E), jnp.float32),
            jax.ShapeDtypeStruct((_TOKENS, _E), jnp.float32),
            jax.ShapeDtypeStruct((_TOKENS, _E), jnp.float32),
            jax.ShapeDtypeStruct((_TOKENS, _E), jnp.float32),
            jax.ShapeDtypeStruct((_TOKENS, 1), jnp.int32),
            jax.ShapeDtypeStruct((1, _LANES), jnp.int32),
            jax.ShapeDtypeStruct((1, _LANES), jnp.int32),
            jax.ShapeDtypeStruct((_TOKENS, _SGW), jnp.float32),
            jax.ShapeDtypeStruct((_TOKENS, _DP), jnp.uint32),
        ],
        scratch_shapes=[pltpu.VMEM((1, _LANES), jnp.float32)],
    )(x, Ws1, bs1r, w2a, b2a, w2b, b2b, w2c, b2c, g, eps)

    (ylog, yidx2, y, zm, zlv, z, rank2, cnt2, start2, sgate,
     xpacked) = router_outs
    rank = rank2.reshape(_TOKENS)
    eid = yidx2.reshape(_TOKENS)
    starts1 = start2.reshape(_LANES)

    xsp, ss, pos = _dispatch(xpacked, rank, eid, starts1, sgate)

    out_sorted = pl.pallas_call(
        _expert_body,
        grid=(_NBLK,),
        in_specs=[
            pl.BlockSpec(memory_space=pltpu.SMEM),
            row(_DP), row(_SGW),
            full(W1), full(b1), full(W2), full(b2),
        ],
        out_specs=row(_D),
        out_shape=jax.ShapeDtypeStruct((_TOKENS, _D), jnp.float32),
    )(cnt2, xsp, ss, W1, b1, W2, b2)

    out = _combine(out_sorted, pos)
    return (out, ylog, eid, y, zm, zlv, z)


# speedup vs baseline: 1.1527x; 1.0431x over previous
import functools

import jax
import jax.numpy as jnp
from jax import lax
from jax.experimental import pallas as pl
from jax.experimental.pallas import tpu as pltpu
from jax.experimental.pallas import tpu_sc as plsc

_TOKENS = 4096
_D = 768
_H = 64
_E = 8
_BLK = 512
_NBLK = _TOKENS // _BLK
_NW = 32
_CHUNK = _TOKENS // _NW
_LANES = 16
_SGW = 128
_DP = _D // 2


def _router_body(x_ref, ws2_ref, bs2_ref, ws1_ref, bs1_ref, g_ref, eps_ref,
                 ylog_ref, yidx_ref, y_ref, zm_ref, zlv_ref, z_ref,
                 meta_ref, cnt_ref, start_ref, sgate_ref, xp_ref, base_ref):
    i = pl.program_id(0)

    @pl.when(i == 0)
    def _init():
        base_ref[...] = jnp.zeros((1, _LANES), jnp.float32)

    xb = x_ref[...]
    h = jnp.maximum(jnp.dot(xb, ws1_ref[...]) + bs1_ref[...], 0.0)
    ctrl = jnp.dot(h, ws2_ref[...]) + bs2_ref[...]
    ylog = ctrl[:, 0:_E]
    zm = ctrl[:, _E:2 * _E]
    zlv = ctrl[:, 2 * _E:3 * _E]
    a = ylog + g_ref[...]
    m = jnp.max(a, axis=1, keepdims=True)
    ex = jnp.exp(a - m)
    ysoft = ex / jnp.sum(ex, axis=1, keepdims=True)
    lane = lax.broadcasted_iota(jnp.int32, (_BLK, _E), 1)
    yidx = jnp.min(jnp.where(a == m, lane, _E), axis=1)
    yhard = (lane == yidx[:, None]).astype(jnp.float32)
    y = ysoft + (yhard - ysoft)
    z = zm + jnp.exp(0.5 * zlv) * eps_ref[...]

    lane16 = lax.broadcasted_iota(jnp.int32, (_BLK, _LANES), 1)
    hot16 = (lane16 == yidx[:, None]).astype(jnp.float32)
    r_io = lax.broadcasted_iota(jnp.int32, (_BLK, _BLK), 0)
    c_io = lax.broadcasted_iota(jnp.int32, (_BLK, _BLK), 1)
    tri = (r_io >= c_io).astype(jnp.bfloat16)
    cum = jnp.dot(tri, hot16.astype(jnp.bfloat16),
                  preferred_element_type=jnp.float32)
    base = base_ref[...]
    rank = jnp.sum((cum - hot16 + base) * hot16, axis=1)
    base = base + cum[_BLK - 1:_BLK, :]
    base_ref[...] = base
    cnt_ref[...] = base.astype(jnp.int32)
    r16 = lax.broadcasted_iota(jnp.int32, (_LANES, _LANES), 0)
    c16 = lax.broadcasted_iota(jnp.int32, (_LANES, _LANES), 1)
    sut = (r16 < c16).astype(jnp.float32)
    starts = jnp.dot(base, sut, precision=lax.Precision.HIGHEST)
    start_ref[...] = starts.astype(jnp.int32)
    s = jnp.sum(yhard * z, axis=1)

    ylog_ref[...] = ylog
    yidx_ref[...] = yidx[:, None]
    y_ref[...] = y
    zm_ref[...] = zm
    zlv_ref[...] = zlv
    z_ref[...] = z
    meta_ref[...] = (rank.astype(jnp.int32) | (yidx << 12))[:, None]
    sgate_ref[...] = jnp.broadcast_to(s[:, None], (_BLK, _SGW))
    xr = lax.bitcast_convert_type(
        xb.astype(jnp.bfloat16).astype(jnp.float32), jnp.uint32)
    xp_ref[...] = (xr[:, :_DP] >> 16) | (xr[:, _DP:] & jnp.uint32(0xFFFF0000))


_SC_MESH = plsc.VectorSubcoreMesh(core_axis_name="c", subcore_axis_name="s")


def _sc_wid():
    return lax.axis_index("s") * 2 + lax.axis_index("c")


@functools.partial(
    pl.kernel,
    out_type=[
        jax.ShapeDtypeStruct((_TOKENS, _DP), jnp.uint32),
        jax.ShapeDtypeStruct((_TOKENS, _SGW), jnp.float32),
        jax.ShapeDtypeStruct((_TOKENS,), jnp.int32),
    ],
    mesh=_SC_MESH,
    scratch_types=[
        pltpu.VMEM((_CHUNK,), jnp.int32),
        pltpu.VMEM((_LANES,), jnp.int32),
        pltpu.VMEM((_CHUNK,), jnp.int32),
        pltpu.VMEM((_CHUNK, _DP), jnp.uint32),
        pltpu.VMEM((_CHUNK, _SGW), jnp.float32),
        pltpu.SemaphoreType.DMA,
        pltpu.SemaphoreType.DMA,
    ],
    compiler_params=pltpu.CompilerParams(needs_layout_passes=False),
)
def _dispatch(xp_hbm, meta_hbm, start_hbm, sg_hbm,
              xsp_hbm, ss_hbm, pos_hbm,
              meta_v, cnt_v, pos_v, rows_v, sg_v, sem_x, sem_sg):
    base = _sc_wid() * _CHUNK
    cp_x = pltpu.async_copy(xp_hbm.at[pl.ds(base, _CHUNK)], rows_v, sem_x)
    cp_sg = pltpu.async_copy(sg_hbm.at[pl.ds(base, _CHUNK)], sg_v, sem_sg)
    pltpu.sync_copy(start_hbm, cnt_v)
    pltpu.sync_copy(meta_hbm.at[pl.ds(base, _CHUNK)], meta_v)
    idx16 = lax.iota(jnp.int32, _LANES)
    sv = cnt_v[...]
    st = [jnp.sum(jnp.where(idx16 == e, sv, 0)) for e in range(_E)]
    for j in range(_CHUNK // _LANES):
        sl = pl.ds(j * _LANES, _LANES)
        mv = meta_v[sl]
        ev = mv >> 12
        acc = jnp.full((_LANES,), 0, jnp.int32)
        for e in range(_E):
            acc = jnp.where(ev == e, st[e], acc)
        pos_v[sl] = (mv & 0xFFF) + acc
    pltpu.sync_copy(pos_v, pos_hbm.at[pl.ds(base, _CHUNK)])
    cp_x.wait()
    pltpu.sync_copy(rows_v, xsp_hbm.at[pos_v])
    cp_sg.wait()
    pltpu.sync_copy(sg_v, ss_hbm.at[pos_v])


@functools.partial(
    pl.kernel,
    out_type=jax.ShapeDtypeStruct((_TOKENS, _D), jnp.float32),
    mesh=_SC_MESH,
    scratch_types=[
        pltpu.VMEM((_CHUNK,), jnp.int32),
        pltpu.VMEM((_CHUNK, _D), jnp.float32),
        pltpu.SemaphoreType.DMA,
    ],
    compiler_params=pltpu.CompilerParams(needs_layout_passes=False),
)
def _combine(os_hbm, pos_hbm, out_hbm, pos_v, rows_v, sem):
    base = _sc_wid() * _CHUNK
    pltpu.sync_copy(pos_hbm.at[pl.ds(base, _CHUNK)], pos_v)
    pltpu.async_copy(os_hbm.at[pos_v], rows_v, sem).wait()
    pltpu.sync_copy(rows_v, out_hbm.at[pl.ds(base, _CHUNK)])


def _expert_body(cnt_ref, xsp_ref, ss_ref, w1_ref, b1_ref, w2_ref,
                 b2_ref, out_ref):
    i = pl.program_id(0)
    pi = xsp_ref[...]
    xlo = lax.bitcast_convert_type(pi << 16, jnp.float32)
    xhi = lax.bitcast_convert_type(pi & jnp.uint32(0xFFFF0000), jnp.float32)
    xf = jnp.concatenate([xlo, xhi], axis=1)
    xbf = xf.astype(jnp.bfloat16)
    out_ref[...] = xf
    sc_col = ss_ref[...][:, 0:1]
    rows = i * _BLK + lax.broadcasted_iota(jnp.int32, (_BLK, 1), 0)
    start = jnp.int32(0)
    for e in range(_E):
        cnt_e = cnt_ref[0, e]
        lo = start
        pred = jnp.logical_and(lo < (i + 1) * _BLK, lo + cnt_e > i * _BLK)

        @pl.when(pred)
        def _run(lo=lo, cnt_e=cnt_e, e=e):
            m = jnp.logical_and(rows >= lo, rows < lo + cnt_e)
            g = m.astype(jnp.float32) * sc_col
            hb = jnp.dot(xbf, w1_ref[e].astype(jnp.bfloat16),
                         preferred_element_type=jnp.float32)
            hb = jnp.maximum(hb + b1_ref[e][None, :], 0.0)
            hg = (hb * g).astype(jnp.bfloat16)
            ob = jnp.dot(hg, w2_ref[e].astype(jnp.bfloat16),
                         preferred_element_type=jnp.float32)
            out_ref[...] += ob + g * b2_ref[e][None, :]

        start = start + cnt_e


def _noise():
    def build():
        key = jax.random.key(42)
        k1, k2 = jax.random.split(key)
        u = jax.random.uniform(k1, (_TOKENS, _E), minval=1e-6, maxval=1.0 - 1e-6)
        g = -jnp.log(-jnp.log(u))
        eps = jax.random.normal(k2, (_TOKENS, _E), dtype=jnp.float32)
        return g, eps
    try:
        with jax.ensure_compile_time_eval():
            return build()
    except Exception:
        return build()


@jax.jit
def kernel(x, Ws1, bs1, Ws2, bs2, W1, b1, W2, b2):
    g, eps = _noise()

    bs2r = bs2.reshape(1, 3 * _E)
    bs1r = bs1.reshape(1, _H)

    row = lambda w: pl.BlockSpec((_BLK, w), lambda i: (i, 0))
    full = lambda arr: pl.BlockSpec(arr.shape, lambda i: (0,) * arr.ndim)

    router_outs = pl.pallas_call(
        _router_body,
        grid=(_NBLK,),
        in_specs=[
            row(_D), full(Ws2), full(bs2r), full(Ws1), full(bs1r),
            row(_E), row(_E),
        ],
        out_specs=[
            row(_E), row(1), row(_E), row(_E), row(_E), row(_E),
            row(1), pl.BlockSpec((1, _LANES), lambda i: (0, 0)),
            pl.BlockSpec((1, _LANES), lambda i: (0, 0)), row(_SGW),
            row(_DP),
        ],
        out_shape=[
            jax.ShapeDtypeStruct((_TOKENS, _E), jnp.float32),
            jax.ShapeDtypeStruct((_TOKENS, 1), jnp.int32),
            jax.ShapeDtypeStruct((_TOKENS, _E), jnp.float32),
            jax.ShapeDtypeStruct((_TOKENS, _E), jnp.float32),
            jax.ShapeDtypeStruct((_TOKENS, _E), jnp.float32),
            jax.ShapeDtypeStruct((_TOKENS, _E), jnp.float32),
            jax.ShapeDtypeStruct((_TOKENS, 1), jnp.int32),
            jax.ShapeDtypeStruct((1, _LANES), jnp.int32),
            jax.ShapeDtypeStruct((1, _LANES), jnp.int32),
            jax.ShapeDtypeStruct((_TOKENS, _SGW), jnp.float32),
            jax.ShapeDtypeStruct((_TOKENS, _DP), jnp.uint32),
        ],
        scratch_shapes=[pltpu.VMEM((1, _LANES), jnp.float32)],
    )(x, Ws2, bs2r, Ws1, bs1r, g, eps)

    (ylog, yidx2, y, zm, zlv, z, meta2, cnt2, start2, sgate,
     xpacked) = router_outs
    meta = meta2.reshape(_TOKENS)
    eid = yidx2.reshape(_TOKENS)
    starts1 = start2.reshape(_LANES)

    xsp, ss, pos = _dispatch(xpacked, meta, starts1, sgate)

    out_sorted = pl.pallas_call(
        _expert_body,
        grid=(_NBLK,),
        in_specs=[
            pl.BlockSpec(memory_space=pltpu.SMEM),
            row(_DP), row(_SGW),
            full(W1), full(b1), full(W2), full(b2),
        ],
        out_specs=row(_D),
        out_shape=jax.ShapeDtypeStruct((_TOKENS, _D), jnp.float32),
    )(cnt2, xsp, ss, W1, b1, W2, b2)

    out = _combine(out_sorted, pos)
    return (out, ylog, eid, y, zm, zlv, z)
